# baseline (speedup 1.0000x reference)
import functools

import jax
import jax.numpy as jnp
from jax import lax
from jax.experimental import pallas as pl
from jax.experimental.pallas import tpu as pltpu
from jax.experimental.pallas import tpu_sc as plsc

N_ROWS = 96
ROW = 512 * 512
K_BOT = 5243
K_TOP = 256900
NC, NS, L = 2, 16, 16
NW = NC * NS
ROWS_PER_W = N_ROWS // NW
CHUNK = 16384
NCHUNK = ROW // CHUNK
HSTRIDE = 4096
HI_OFF = 2048
CAP = 24576
BIG = 1 << 30


def _monotone_u32_vec(f):
    i = lax.bitcast_convert_type(f, jnp.int32)
    flip = lax.shift_right_arithmetic(i, 31) & jnp.int32(0x7FFFFFFF)
    return lax.bitcast_convert_type(i ^ flip, jnp.uint32)


def _u32_to_f32(u):
    i = lax.bitcast_convert_type(u, jnp.int32)
    flip = lax.shift_right_arithmetic(i, 31) & jnp.int32(0x7FFFFFFF)
    return lax.bitcast_convert_type(i ^ flip, jnp.float32)


def _sc_select_kernel(x_hbm, out_hbm, buf0, buf1, hist, cand, res,
                      sem0, sem1):
    wid = lax.axis_index("s") * NC + lax.axis_index("c")
    lanes = lax.broadcasted_iota(jnp.int32, (L,), 0)
    ones_i = jnp.ones((L,), jnp.int32)
    zeros_i = jnp.zeros((L,), jnp.int32)
    true_m = jnp.full((L,), True)

    def start(row, c, buf, sem):
        pltpu.make_async_copy(
            x_hbm.at[row, pl.ds(c * CHUNK, CHUNK)], buf, sem).start()

    def wait(row, buf, sem):
        pltpu.make_async_copy(
            x_hbm.at[row, pl.ds(0, CHUNK)], buf, sem).wait()

    def zero_hist(nwords):
        @plsc.parallel_loop(0, nwords // L, unroll=8)
        def _(i):
            hist[pl.ds(i * L, L)] = zeros_i

    def preprime(row):
        start(row, 0, buf0, sem0)
        start(row, 1, buf1, sem1)

    def stream_row(row, process, carry, preprimed=False):
        if not preprimed:
            preprime(row)

        def cbody(g, carry):
            c0 = 2 * g
            wait(row, buf0, sem0)
            carry = process(buf0, carry)

            @pl.when(c0 + 2 < NCHUNK)
            def _():
                start(row, c0 + 2, buf0, sem0)
            wait(row, buf1, sem1)
            carry = process(buf1, carry)

            @pl.when(c0 + 3 < NCHUNK)
            def _():
                start(row, c0 + 3, buf1, sem1)
            return carry

        return lax.fori_loop(0, NCHUNK // 2, cbody, carry)

    def hist_update(ukey, shift, nbits, p_lo, p_hi, stride, valid=None):
        digit = (lax.shift_right_logical(ukey, jnp.uint32(shift))
                 .astype(jnp.int32)) & jnp.int32((1 << nbits) - 1)
        m_lo = (ukey ^ p_lo) < jnp.uint32(1 << (shift + nbits))
        m_hi = (ukey ^ p_hi) < jnp.uint32(1 << (shift + nbits))
        if valid is not None:
            m_lo = jnp.logical_and(m_lo, valid)
            m_hi = jnp.logical_and(m_hi, valid)
        idx = lanes * stride + digit + jnp.where(
            m_hi, jnp.int32(stride // 2), jnp.int32(0))
        plsc.addupdate_scatter(hist, [idx], ones_i,
                               mask=jnp.logical_or(m_lo, m_hi))

    def scan_region(base, kplus1, ndig, stride):
        def sbody(g, carry):
            csum, cnt, below, upper = carry
            acc = zeros_i
            for l in range(L):
                acc = acc + hist[pl.ds(l * stride + base + g * L, L)]
            pre = plsc.cumsum(acc) + csum
            lt = pre < kplus1
            cnt = cnt + jnp.sum(lt.astype(jnp.int32))
            below = jnp.maximum(below, jnp.max(jnp.where(lt, pre, 0)))
            upper = jnp.minimum(upper, jnp.min(
                jnp.where(lt, jnp.int32(BIG), pre)))
            csum = csum + jnp.sum(acc)
            return csum, cnt, below, upper
        _, digit, below, upper = lax.fori_loop(
            0, ndig // L, sbody,
            (jnp.int32(0), jnp.int32(0), jnp.int32(0), jnp.int32(BIG)))
        return digit, below, upper - below

    def scan_both(p_lo, p_hi, cb_lo, cb_hi, shift, nbits, stride):
        ndig = 1 << nbits
        if shift + nbits == 32:
            base_lo = jnp.int32(0)
            base_hi = jnp.int32(0)
        else:
            eq = (p_lo >> jnp.uint32(shift + nbits)) == (
                p_hi >> jnp.uint32(shift + nbits))
            base_lo = jnp.where(eq, jnp.int32(stride // 2), jnp.int32(0))
            base_hi = jnp.int32(stride // 2)
        d_lo, below_lo, bk_lo = scan_region(
            base_lo, K_BOT + 1 - cb_lo, ndig, stride)
        d_hi, below_hi, bk_hi = scan_region(
            base_hi, K_TOP + 1 - cb_hi, ndig, stride)
        p_lo = p_lo | lax.shift_left(d_lo.astype(jnp.uint32),
                                     jnp.uint32(shift))
        p_hi = p_hi | lax.shift_left(d_hi.astype(jnp.uint32),
                                     jnp.uint32(shift))
        return p_lo, p_hi, cb_lo + below_lo, cb_hi + below_hi, bk_lo, bk_hi

    def row_body(r, _):
        row = wid * ROWS_PER_W + r

        zero_hist(2048 * L)

        def p0(buf, carry):
            @plsc.parallel_loop(0, CHUNK // L, unroll=24, carry=carry)
            def mm(v, carry):
                mnv, mxv = carry
                f = buf[pl.ds(v * L, L)]
                ukey = _monotone_u32_vec(f)
                digit = lax.shift_right_logical(
                    ukey, jnp.uint32(21)).astype(jnp.int32)
                plsc.addupdate_scatter(hist, [lanes * 2048 + digit],
                                       ones_i, mask=true_m)
                return jnp.minimum(mnv, f), jnp.maximum(mxv, f)
            return mm

        minv, maxv = stream_row(row, p0, (
            jnp.full((L,), jnp.float32(jnp.inf)),
            jnp.full((L,), -jnp.float32(jnp.inf))))
        preprime(row)

        def s0body(g, carry):
            (csum, cnt1, below1, upper1, cnt2, below2, upper2) = carry
            gw = ((g + 64) & 127) * L
            acc = zeros_i
            for l in range(L):
                acc = acc + hist[pl.ds(l * 2048 + gw, L)]
            pre = plsc.cumsum(acc) + csum
            lt1 = pre < K_BOT + 1
            cnt1 = cnt1 + jnp.sum(lt1.astype(jnp.int32))
            below1 = jnp.maximum(below1, jnp.max(jnp.where(lt1, pre, 0)))
            upper1 = jnp.minimum(upper1, jnp.min(
                jnp.where(lt1, jnp.int32(BIG), pre)))
            lt2 = pre < K_TOP + 1
            cnt2 = cnt2 + jnp.sum(lt2.astype(jnp.int32))
            below2 = jnp.maximum(below2, jnp.max(jnp.where(lt2, pre, 0)))
            upper2 = jnp.minimum(upper2, jnp.min(
                jnp.where(lt2, jnp.int32(BIG), pre)))
            csum = csum + jnp.sum(acc)
            return (csum, cnt1, below1, upper1, cnt2, below2, upper2)

        z = jnp.int32(0)
        (_, c1, cb_lo, u1, c2, cb_hi, u2) = lax.fori_loop(
            0, 128, s0body, (z, z, z, jnp.int32(BIG), z, z, jnp.int32(BIG)))
        bk_lo = u1 - cb_lo
        bk_hi = u2 - cb_hi
        d_lo = (1024 + c1) & 2047
        d_hi = (1024 + c2) & 2047
        p_lo = lax.shift_left(d_lo.astype(jnp.uint32), jnp.uint32(21))
        p_hi = lax.shift_left(d_hi.astype(jnp.uint32), jnp.uint32(21))

        eq11 = p_lo == p_hi
        n_cand = jnp.where(eq11, bk_lo, bk_lo + bk_hi)
        fits = n_cand <= CAP

        def cand_chain(p_lo, p_hi, cb_lo, cb_hi):
            def pc(buf, off):
                @plsc.parallel_loop(0, CHUNK // L, unroll=24, carry=off)
                def loop(v, off):
                    f = buf[pl.ds(v * L, L)]
                    ukey = _monotone_u32_vec(f)
                    m_lo = (ukey ^ p_lo) < jnp.uint32(1 << 21)
                    m_hi = (ukey ^ p_hi) < jnp.uint32(1 << 21)
                    m = jnp.logical_or(m_lo, m_hi)
                    pos = off + plsc.cumsum(m.astype(jnp.int32))
                    plsc.store_scatter(
                        cand, [pos],
                        lax.bitcast_convert_type(ukey, jnp.int32), mask=m)
                    return off + plsc.all_reduce_population_count(m)
                return loop
            stream_row(row, pc, jnp.full((L,), -1, jnp.int32),
                       preprimed=True)

            nvec = (n_cand + (L - 1)) >> 4
            for shift in (14, 7, 0):
                zero_hist(256 * L)

                @plsc.parallel_loop(0, nvec, unroll=4)
                def _(v, shift=shift, p_lo=p_lo, p_hi=p_hi):
                    ukey = lax.bitcast_convert_type(
                        cand[pl.ds(v * L, L)], jnp.uint32)
                    inb = (v * L + lanes) < n_cand
                    hist_update(ukey, shift, 7, p_lo, p_hi, 256, valid=inb)

                p_lo, p_hi, cb_lo, cb_hi, _, _ = scan_both(
                    p_lo, p_hi, cb_lo, cb_hi, shift, 7, 256)
            return p_lo, p_hi, cb_lo, cb_hi

        def full_chain(p_lo, p_hi, cb_lo, cb_hi):
            first = True
            for shift, nbits in ((10, 11), (0, 10)):
                zero_hist(4096 * L)

                def pf(buf, carry, shift=shift, nbits=nbits,
                       p_lo=p_lo, p_hi=p_hi):
                    @plsc.parallel_loop(0, CHUNK // L, unroll=8)
                    def _(v):
                        ukey = _monotone_u32_vec(buf[pl.ds(v * L, L)])
                        hist_update(ukey, shift, nbits, p_lo, p_hi, 4096)
                    return carry
                stream_row(row, pf, jnp.int32(0), preprimed=first)
                first = False
                p_lo, p_hi, cb_lo, cb_hi, _, _ = scan_both(
                    p_lo, p_hi, cb_lo, cb_hi, shift, nbits, 4096)
            return p_lo, p_hi, cb_lo, cb_hi

        p_lo, p_hi, cb_lo, cb_hi = lax.cond(
            fits, cand_chain, full_chain, p_lo, p_hi, cb_lo, cb_hi)

        bot = _u32_to_f32(p_lo)
        top = _u32_to_f32(p_hi)
        mn = jnp.min(minv)
        mx = jnp.max(maxv)
        vec = jnp.where(lanes == 0, bot,
              jnp.where(lanes == 1, top,
              jnp.where(lanes == 2, mn,
              jnp.where(lanes == 3, mx, jnp.float32(0.0)))))
        res[pl.ds(r * L, L)] = vec
        return 0

    lax.fori_loop(0, ROWS_PER_W, row_body, 0)
    pltpu.sync_copy(res, out_hbm.at[pl.ds(wid * ROWS_PER_W * L,
                                          ROWS_PER_W * L)])


def _sc_select(x2d):
    mesh = plsc.VectorSubcoreMesh(core_axis_name="c", subcore_axis_name="s",
                                  num_cores=NC)
    run = functools.partial(
        pl.kernel,
        mesh=mesh,
        compiler_params=pltpu.CompilerParams(needs_layout_passes=False),
        out_type=jax.ShapeDtypeStruct((N_ROWS * L,), jnp.float32),
        scratch_types=[
            pltpu.VMEM((CHUNK,), jnp.float32),
            pltpu.VMEM((CHUNK,), jnp.float32),
            pltpu.VMEM((L * HSTRIDE,), jnp.int32),
            pltpu.VMEM((CAP,), jnp.int32),
            pltpu.VMEM((ROWS_PER_W * L,), jnp.float32),
            pltpu.SemaphoreType.DMA,
            pltpu.SemaphoreType.DMA,
        ],
    )(_sc_select_kernel)
    return run(x2d)


def _normalize_body(stats_all_ref, x_ref, stats_row_ref, out_ref):
    s = stats_all_ref[:, 0, :]
    bot_raw, top_raw = s[:, 0], s[:, 1]
    mn, mx = s[:, 2], s[:, 3]
    same = top_raw == bot_raw
    top1 = jnp.where(same, mx, top_raw)
    bot1 = jnp.where(same, mn, bot_raw)
    all_black = jnp.any(top1 == 0.0)
    all_const = jnp.any(top1 == bot1)
    b_row = stats_row_ref[:, 0, 0]
    t_row = stats_row_ref[:, 0, 1]
    same_r = t_row == b_row
    t1 = jnp.where(same_r, stats_row_ref[:, 0, 3], t_row)
    b1 = jnp.where(same_r, stats_row_ref[:, 0, 2], b_row)
    t = jnp.where(all_black, jnp.float32(1.0), t1)
    b = jnp.where(jnp.logical_and(jnp.logical_not(all_black), all_const),
                  jnp.float32(0.0), b1)
    scale = jnp.float32(1.0) / (t - b)
    out_ref[...] = jnp.clip(
        (x_ref[...] - b[:, None, None]) * scale[:, None, None], 0.0, 1.0)


RB = 8


def kernel(x):
    stats = _sc_select(x.reshape(N_ROWS, ROW)).reshape(N_ROWS, 1, L)

    out = pl.pallas_call(
        _normalize_body,
        grid=(N_ROWS // RB,),
        in_specs=[
            pl.BlockSpec((N_ROWS, 1, L), lambda i: (0, 0, 0)),
            pl.BlockSpec((RB, 512, 512), lambda i: (i, 0, 0)),
            pl.BlockSpec((RB, 1, L), lambda i: (i, 0, 0)),
        ],
        out_specs=pl.BlockSpec((RB, 512, 512), lambda i: (i, 0, 0)),
        out_shape=jax.ShapeDtypeStruct((N_ROWS, 512, 512), jnp.float32),
    )(stats, x, stats)
    return out


# speedup vs baseline: 1.8051x; 1.8051x over previous
import functools

import jax
import jax.numpy as jnp
from jax import lax
from jax.experimental import pallas as pl
from jax.experimental.pallas import tpu as pltpu
from jax.experimental.pallas import tpu_sc as plsc

N_ROWS = 96
ROW = 512 * 512
K_BOT = 5243
K_TOP = 256900
NC, NS, L = 2, 16, 16
NW = NC * NS
ROWS_PER_W = N_ROWS // NW
CHUNK = 16384
NCHUNK = ROW // CHUNK
HSTRIDE = 4096
HI_OFF = 2048
CAP = 24576
BIG = 1 << 30


def _monotone_u32_vec(f):
    i = lax.bitcast_convert_type(f, jnp.int32)
    flip = lax.shift_right_arithmetic(i, 31) & jnp.int32(0x7FFFFFFF)
    return lax.bitcast_convert_type(i ^ flip, jnp.uint32)


def _u32_to_f32(u):
    i = lax.bitcast_convert_type(u, jnp.int32)
    flip = lax.shift_right_arithmetic(i, 31) & jnp.int32(0x7FFFFFFF)
    return lax.bitcast_convert_type(i ^ flip, jnp.float32)


def _sc_select_kernel(x_hbm, out_hbm, buf0, buf1, hist, cand, res,
                      sem0, sem1):
    wid = lax.axis_index("s") * NC + lax.axis_index("c")
    lanes = lax.broadcasted_iota(jnp.int32, (L,), 0)
    ones_i = jnp.ones((L,), jnp.int32)
    zeros_i = jnp.zeros((L,), jnp.int32)
    true_m = jnp.full((L,), True)

    def start(row, c, buf, sem):
        pltpu.make_async_copy(
            x_hbm.at[row, pl.ds(c * CHUNK, CHUNK)], buf, sem).start()

    def wait(row, buf, sem):
        pltpu.make_async_copy(
            x_hbm.at[row, pl.ds(0, CHUNK)], buf, sem).wait()

    def zero_hist(nwords):
        @plsc.parallel_loop(0, nwords // L, unroll=8)
        def _(i):
            hist[pl.ds(i * L, L)] = zeros_i

    def preprime(row):
        start(row, 0, buf0, sem0)
        start(row, 1, buf1, sem1)

    def stream_row(row, process, carry, preprimed=False):
        if not preprimed:
            preprime(row)

        def cbody(g, carry):
            c0 = 2 * g
            wait(row, buf0, sem0)
            carry = process(buf0, carry)

            @pl.when(c0 + 2 < NCHUNK)
            def _():
                start(row, c0 + 2, buf0, sem0)
            wait(row, buf1, sem1)
            carry = process(buf1, carry)

            @pl.when(c0 + 3 < NCHUNK)
            def _():
                start(row, c0 + 3, buf1, sem1)
            return carry

        return lax.fori_loop(0, NCHUNK // 2, cbody, carry)

    def hist_update(ukey, shift, nbits, p_lo, p_hi, stride, valid=None):
        digit = (lax.shift_right_logical(ukey, jnp.uint32(shift))
                 .astype(jnp.int32)) & jnp.int32((1 << nbits) - 1)
        m_lo = (ukey ^ p_lo) < jnp.uint32(1 << (shift + nbits))
        m_hi = (ukey ^ p_hi) < jnp.uint32(1 << (shift + nbits))
        if valid is not None:
            m_lo = jnp.logical_and(m_lo, valid)
            m_hi = jnp.logical_and(m_hi, valid)
        idx = lanes * stride + digit + jnp.where(
            m_hi, jnp.int32(stride // 2), jnp.int32(0))
        plsc.addupdate_scatter(hist, [idx], ones_i,
                               mask=jnp.logical_or(m_lo, m_hi))

    def scan_region(base, kplus1, ndig, stride):
        def sbody(g, carry):
            csum, cnt, below, upper = carry
            acc = zeros_i
            for l in range(L):
                acc = acc + hist[pl.ds(l * stride + base + g * L, L)]
            pre = plsc.cumsum(acc) + csum
            lt = pre < kplus1
            cnt = cnt + jnp.sum(lt.astype(jnp.int32))
            below = jnp.maximum(below, jnp.max(jnp.where(lt, pre, 0)))
            upper = jnp.minimum(upper, jnp.min(
                jnp.where(lt, jnp.int32(BIG), pre)))
            csum = csum + jnp.sum(acc)
            return csum, cnt, below, upper
        _, digit, below, upper = lax.fori_loop(
            0, ndig // L, sbody,
            (jnp.int32(0), jnp.int32(0), jnp.int32(0), jnp.int32(BIG)))
        return digit, below, upper - below

    def scan_both(p_lo, p_hi, cb_lo, cb_hi, shift, nbits, stride):
        ndig = 1 << nbits
        if shift + nbits == 32:
            base_lo = jnp.int32(0)
            base_hi = jnp.int32(0)
        else:
            eq = (p_lo >> jnp.uint32(shift + nbits)) == (
                p_hi >> jnp.uint32(shift + nbits))
            base_lo = jnp.where(eq, jnp.int32(stride // 2), jnp.int32(0))
            base_hi = jnp.int32(stride // 2)
        d_lo, below_lo, bk_lo = scan_region(
            base_lo, K_BOT + 1 - cb_lo, ndig, stride)
        d_hi, below_hi, bk_hi = scan_region(
            base_hi, K_TOP + 1 - cb_hi, ndig, stride)
        p_lo = p_lo | lax.shift_left(d_lo.astype(jnp.uint32),
                                     jnp.uint32(shift))
        p_hi = p_hi | lax.shift_left(d_hi.astype(jnp.uint32),
                                     jnp.uint32(shift))
        return p_lo, p_hi, cb_lo + below_lo, cb_hi + below_hi, bk_lo, bk_hi

    def row_body(r, _):
        row = wid * ROWS_PER_W + r

        zero_hist(2048 * L)

        def p0(buf, carry):
            @plsc.parallel_loop(0, CHUNK // L, unroll=16, carry=carry)
            def mm(v, carry):
                mnv, mxv = carry
                f = buf[pl.ds(v * L, L)]
                ukey = _monotone_u32_vec(f)
                digit = lax.shift_right_logical(
                    ukey, jnp.uint32(21)).astype(jnp.int32)
                plsc.addupdate_scatter(hist, [lanes * 2048 + digit],
                                       ones_i, mask=true_m)
                return jnp.minimum(mnv, f), jnp.maximum(mxv, f)
            return mm

        minv, maxv = stream_row(row, p0, (
            jnp.full((L,), jnp.float32(jnp.inf)),
            jnp.full((L,), -jnp.float32(jnp.inf))))
        preprime(row)

        def s0body(g, carry):
            (csum, cnt1, below1, upper1, cnt2, below2, upper2) = carry
            gw = ((g + 64) & 127) * L
            acc = zeros_i
            for l in range(L):
                acc = acc + hist[pl.ds(l * 2048 + gw, L)]
            pre = plsc.cumsum(acc) + csum
            lt1 = pre < K_BOT + 1
            cnt1 = cnt1 + jnp.sum(lt1.astype(jnp.int32))
            below1 = jnp.maximum(below1, jnp.max(jnp.where(lt1, pre, 0)))
            upper1 = jnp.minimum(upper1, jnp.min(
                jnp.where(lt1, jnp.int32(BIG), pre)))
            lt2 = pre < K_TOP + 1
            cnt2 = cnt2 + jnp.sum(lt2.astype(jnp.int32))
            below2 = jnp.maximum(below2, jnp.max(jnp.where(lt2, pre, 0)))
            upper2 = jnp.minimum(upper2, jnp.min(
                jnp.where(lt2, jnp.int32(BIG), pre)))
            csum = csum + jnp.sum(acc)
            return (csum, cnt1, below1, upper1, cnt2, below2, upper2)

        z = jnp.int32(0)
        (_, c1, cb_lo, u1, c2, cb_hi, u2) = lax.fori_loop(
            0, 128, s0body, (z, z, z, jnp.int32(BIG), z, z, jnp.int32(BIG)))
        bk_lo = u1 - cb_lo
        bk_hi = u2 - cb_hi
        d_lo = (1024 + c1) & 2047
        d_hi = (1024 + c2) & 2047
        p_lo = lax.shift_left(d_lo.astype(jnp.uint32), jnp.uint32(21))
        p_hi = lax.shift_left(d_hi.astype(jnp.uint32), jnp.uint32(21))

        eq11 = p_lo == p_hi
        n_cand = jnp.where(eq11, bk_lo, bk_lo + bk_hi)
        fits = n_cand <= CAP

        def cand_chain(p_lo, p_hi, cb_lo, cb_hi):
            def pc(buf, off):
                @plsc.parallel_loop(0, CHUNK // L, unroll=16, carry=off)
                def loop(v, off):
                    f = buf[pl.ds(v * L, L)]
                    ukey = _monotone_u32_vec(f)
                    m_lo = (ukey ^ p_lo) < jnp.uint32(1 << 21)
                    m_hi = (ukey ^ p_hi) < jnp.uint32(1 << 21)
                    m = jnp.logical_or(m_lo, m_hi)
                    pos = off + plsc.cumsum(m.astype(jnp.int32))
                    plsc.store_scatter(
                        cand, [pos],
                        lax.bitcast_convert_type(ukey, jnp.int32), mask=m)
                    return off + plsc.all_reduce_population_count(m)
                return loop
            stream_row(row, pc, jnp.full((L,), -1, jnp.int32),
                       preprimed=True)

            nvec = (n_cand + (L - 1)) >> 4
            for shift in (14, 7, 0):
                zero_hist(256 * L)

                @plsc.parallel_loop(0, nvec, unroll=4)
                def _(v, shift=shift, p_lo=p_lo, p_hi=p_hi):
                    ukey = lax.bitcast_convert_type(
                        cand[pl.ds(v * L, L)], jnp.uint32)
                    inb = (v * L + lanes) < n_cand
                    hist_update(ukey, shift, 7, p_lo, p_hi, 256, valid=inb)

                p_lo, p_hi, cb_lo, cb_hi, _, _ = scan_both(
                    p_lo, p_hi, cb_lo, cb_hi, shift, 7, 256)
            return p_lo, p_hi, cb_lo, cb_hi

        def full_chain(p_lo, p_hi, cb_lo, cb_hi):
            first = True
            for shift, nbits in ((10, 11), (0, 10)):
                zero_hist(4096 * L)

                def pf(buf, carry, shift=shift, nbits=nbits,
                       p_lo=p_lo, p_hi=p_hi):
                    @plsc.parallel_loop(0, CHUNK // L, unroll=8)
                    def _(v):
                        ukey = _monotone_u32_vec(buf[pl.ds(v * L, L)])
                        hist_update(ukey, shift, nbits, p_lo, p_hi, 4096)
                    return carry
                stream_row(row, pf, jnp.int32(0), preprimed=first)
                first = False
                p_lo, p_hi, cb_lo, cb_hi, _, _ = scan_both(
                    p_lo, p_hi, cb_lo, cb_hi, shift, nbits, 4096)
            return p_lo, p_hi, cb_lo, cb_hi

        p_lo, p_hi, cb_lo, cb_hi = lax.cond(
            fits, cand_chain, full_chain, p_lo, p_hi, cb_lo, cb_hi)

        bot = _u32_to_f32(p_lo)
        top = _u32_to_f32(p_hi)
        mn = jnp.min(minv)
        mx = jnp.max(maxv)
        vec = jnp.where(lanes == 0, bot,
              jnp.where(lanes == 1, top,
              jnp.where(lanes == 2, mn,
              jnp.where(lanes == 3, mx, jnp.float32(0.0)))))
        res[pl.ds(r * L, L)] = vec
        return 0

    lax.fori_loop(0, ROWS_PER_W, row_body, 0)
    pltpu.sync_copy(res, out_hbm.at[pl.ds(wid * ROWS_PER_W * L,
                                          ROWS_PER_W * L)])


def _sc_select(x2d):
    mesh = plsc.VectorSubcoreMesh(core_axis_name="c", subcore_axis_name="s",
                                  num_cores=NC)
    run = functools.partial(
        pl.kernel,
        mesh=mesh,
        compiler_params=pltpu.CompilerParams(needs_layout_passes=False),
        out_type=jax.ShapeDtypeStruct((N_ROWS * L,), jnp.float32),
        scratch_types=[
            pltpu.VMEM((CHUNK,), jnp.float32),
            pltpu.VMEM((CHUNK,), jnp.float32),
            pltpu.VMEM((L * HSTRIDE,), jnp.int32),
            pltpu.VMEM((CAP,), jnp.int32),
            pltpu.VMEM((ROWS_PER_W * L,), jnp.float32),
            pltpu.SemaphoreType.DMA,
            pltpu.SemaphoreType.DMA,
        ],
    )(_sc_select_kernel)
    return run(x2d)


def _normalize_body(stats_all_ref, x_ref, stats_row_ref, out_ref):
    s = stats_all_ref[:, 0, :]
    bot_raw, top_raw = s[:, 0], s[:, 1]
    mn, mx = s[:, 2], s[:, 3]
    same = top_raw == bot_raw
    top1 = jnp.where(same, mx, top_raw)
    bot1 = jnp.where(same, mn, bot_raw)
    all_black = jnp.any(top1 == 0.0)
    all_const = jnp.any(top1 == bot1)
    b_row = stats_row_ref[:, 0, 0]
    t_row = stats_row_ref[:, 0, 1]
    same_r = t_row == b_row
    t1 = jnp.where(same_r, stats_row_ref[:, 0, 3], t_row)
    b1 = jnp.where(same_r, stats_row_ref[:, 0, 2], b_row)
    t = jnp.where(all_black, jnp.float32(1.0), t1)
    b = jnp.where(jnp.logical_and(jnp.logical_not(all_black), all_const),
                  jnp.float32(0.0), b1)
    scale = jnp.float32(1.0) / (t - b)
    out_ref[...] = jnp.clip(
        (x_ref[...] - b[:, None, None]) * scale[:, None, None], 0.0, 1.0)


RB = 8


def kernel(x):
    stats = _sc_select(x.reshape(N_ROWS, ROW)).reshape(N_ROWS, 1, L)

    out = pl.pallas_call(
        _normalize_body,
        grid=(N_ROWS // RB,),
        in_specs=[
            pl.BlockSpec((N_ROWS, 1, L), lambda i: (0, 0, 0)),
            pl.BlockSpec((RB, 512, 512), lambda i: (i, 0, 0)),
            pl.BlockSpec((RB, 1, L), lambda i: (i, 0, 0)),
        ],
        out_specs=pl.BlockSpec((RB, 512, 512), lambda i: (i, 0, 0)),
        out_shape=jax.ShapeDtypeStruct((N_ROWS, 512, 512), jnp.float32),
    )(stats, x, stats)
    return out


# speedup vs baseline: 1.8057x; 1.0004x over previous
import functools

import jax
import jax.numpy as jnp
from jax import lax
from jax.experimental import pallas as pl
from jax.experimental.pallas import tpu as pltpu
from jax.experimental.pallas import tpu_sc as plsc

N_ROWS = 96
ROW = 512 * 512
K_BOT = 5243
K_TOP = 256900
NC, NS, L = 2, 16, 16
NW = NC * NS
ROWS_PER_W = N_ROWS // NW
CHUNK = 16384
NCHUNK = ROW // CHUNK
HWORDS = 65536
CAP = 24576
BIG = 1 << 30


def _monotone_u32_vec(f):
    i = lax.bitcast_convert_type(f, jnp.int32)
    flip = lax.shift_right_arithmetic(i, 31) & jnp.int32(0x7FFFFFFF)
    return lax.bitcast_convert_type(i ^ flip, jnp.uint32)


def _u32_to_f32(u):
    i = lax.bitcast_convert_type(u, jnp.int32)
    flip = lax.shift_right_arithmetic(i, 31) & jnp.int32(0x7FFFFFFF)
    return lax.bitcast_convert_type(i ^ flip, jnp.float32)


def _sc_select_kernel(x_hbm, out_hbm, buf0, buf1, hist, cand, res,
                      sem0, sem1):
    wid = lax.axis_index("s") * NC + lax.axis_index("c")
    lanes = lax.broadcasted_iota(jnp.int32, (L,), 0)
    ones_i = jnp.ones((L,), jnp.int32)
    zeros_i = jnp.zeros((L,), jnp.int32)
    true_m = jnp.full((L,), True)

    def start(row, c, buf, sem):
        pltpu.make_async_copy(
            x_hbm.at[row, pl.ds(c * CHUNK, CHUNK)], buf, sem).start()

    def wait(row, buf, sem):
        pltpu.make_async_copy(
            x_hbm.at[row, pl.ds(0, CHUNK)], buf, sem).wait()

    def zero_hist(nwords):
        @plsc.parallel_loop(0, nwords // L, unroll=8)
        def _(i):
            hist[pl.ds(i * L, L)] = zeros_i

    def preprime(row):
        start(row, 0, buf0, sem0)
        start(row, 1, buf1, sem1)

    def stream_row(row, process, carry, preprimed=False):
        if not preprimed:
            preprime(row)

        def cbody(g, carry):
            c0 = 2 * g
            wait(row, buf0, sem0)
            carry = process(buf0, carry)

            @pl.when(c0 + 2 < NCHUNK)
            def _():
                start(row, c0 + 2, buf0, sem0)
            wait(row, buf1, sem1)
            carry = process(buf1, carry)

            @pl.when(c0 + 3 < NCHUNK)
            def _():
                start(row, c0 + 3, buf1, sem1)
            return carry

        return lax.fori_loop(0, NCHUNK // 2, cbody, carry)

    def hist_update(ukey, shift, nbits, p_lo, p_hi, stride, valid=None):
        digit = (lax.shift_right_logical(ukey, jnp.uint32(shift))
                 .astype(jnp.int32)) & jnp.int32((1 << nbits) - 1)
        m_lo = (ukey ^ p_lo) < jnp.uint32(1 << (shift + nbits))
        m_hi = (ukey ^ p_hi) < jnp.uint32(1 << (shift + nbits))
        if valid is not None:
            m_lo = jnp.logical_and(m_lo, valid)
            m_hi = jnp.logical_and(m_hi, valid)
        idx = lanes * stride + digit + jnp.where(
            m_hi, jnp.int32(stride // 2), jnp.int32(0))
        plsc.addupdate_scatter(hist, [idx], ones_i,
                               mask=jnp.logical_or(m_lo, m_hi))

    def scan_region(base, kplus1, ndig, stride):
        def sbody(g, carry):
            csum, cnt, below, upper = carry
            acc = zeros_i
            for l in range(L):
                acc = acc + hist[pl.ds(l * stride + base + g * L, L)]
            pre = plsc.cumsum(acc) + csum
            lt = pre < kplus1
            cnt = cnt + jnp.sum(lt.astype(jnp.int32))
            below = jnp.maximum(below, jnp.max(jnp.where(lt, pre, 0)))
            upper = jnp.minimum(upper, jnp.min(
                jnp.where(lt, jnp.int32(BIG), pre)))
            csum = csum + jnp.sum(acc)
            return csum, cnt, below, upper
        _, digit, below, upper = lax.fori_loop(
            0, ndig // L, sbody,
            (jnp.int32(0), jnp.int32(0), jnp.int32(0), jnp.int32(BIG)))
        return digit, below, upper - below

    def scan_both(p_lo, p_hi, cb_lo, cb_hi, shift, nbits, stride):
        ndig = 1 << nbits
        if shift + nbits == 32:
            base_lo = jnp.int32(0)
            base_hi = jnp.int32(0)
        else:
            eq = (p_lo >> jnp.uint32(shift + nbits)) == (
                p_hi >> jnp.uint32(shift + nbits))
            base_lo = jnp.where(eq, jnp.int32(stride // 2), jnp.int32(0))
            base_hi = jnp.int32(stride // 2)
        d_lo, below_lo, bk_lo = scan_region(
            base_lo, K_BOT + 1 - cb_lo, ndig, stride)
        d_hi, below_hi, bk_hi = scan_region(
            base_hi, K_TOP + 1 - cb_hi, ndig, stride)
        p_lo = p_lo | lax.shift_left(d_lo.astype(jnp.uint32),
                                     jnp.uint32(shift))
        p_hi = p_hi | lax.shift_left(d_hi.astype(jnp.uint32),
                                     jnp.uint32(shift))
        return p_lo, p_hi, cb_lo + below_lo, cb_hi + below_hi, bk_lo, bk_hi

    def row_body(r, _):
        row = wid * ROWS_PER_W + r

        zero_hist(2048 * L)

        def p0(buf, carry):
            @plsc.parallel_loop(0, CHUNK // L, unroll=16, carry=carry)
            def mm(v, carry):
                mnv, mxv = carry
                f = buf[pl.ds(v * L, L)]
                ukey = _monotone_u32_vec(f)
                digit = lax.shift_right_logical(
                    ukey, jnp.uint32(21)).astype(jnp.int32)
                plsc.addupdate_scatter(hist, [lanes * 2048 + digit],
                                       ones_i, mask=true_m)
                return jnp.minimum(mnv, f), jnp.maximum(mxv, f)
            return mm

        minv, maxv = stream_row(row, p0, (
            jnp.full((L,), jnp.float32(jnp.inf)),
            jnp.full((L,), -jnp.float32(jnp.inf))))
        preprime(row)

        def s0body(g, carry):
            (csum, cnt1, below1, upper1, cnt2, below2, upper2) = carry
            gw = ((g + 64) & 127) * L
            acc = zeros_i
            for l in range(L):
                acc = acc + hist[pl.ds(l * 2048 + gw, L)]
            pre = plsc.cumsum(acc) + csum
            lt1 = pre < K_BOT + 1
            cnt1 = cnt1 + jnp.sum(lt1.astype(jnp.int32))
            below1 = jnp.maximum(below1, jnp.max(jnp.where(lt1, pre, 0)))
            upper1 = jnp.minimum(upper1, jnp.min(
                jnp.where(lt1, jnp.int32(BIG), pre)))
            lt2 = pre < K_TOP + 1
            cnt2 = cnt2 + jnp.sum(lt2.astype(jnp.int32))
            below2 = jnp.maximum(below2, jnp.max(jnp.where(lt2, pre, 0)))
            upper2 = jnp.minimum(upper2, jnp.min(
                jnp.where(lt2, jnp.int32(BIG), pre)))
            csum = csum + jnp.sum(acc)
            return (csum, cnt1, below1, upper1, cnt2, below2, upper2)

        z = jnp.int32(0)
        (_, c1, cb_lo, u1, c2, cb_hi, u2) = lax.fori_loop(
            0, 128, s0body, (z, z, z, jnp.int32(BIG), z, z, jnp.int32(BIG)))
        bk_lo = u1 - cb_lo
        bk_hi = u2 - cb_hi
        d_lo = (1024 + c1) & 2047
        d_hi = (1024 + c2) & 2047
        p_lo = lax.shift_left(d_lo.astype(jnp.uint32), jnp.uint32(21))
        p_hi = lax.shift_left(d_hi.astype(jnp.uint32), jnp.uint32(21))

        eq11 = p_lo == p_hi
        n_cand = jnp.where(eq11, bk_lo, bk_lo + bk_hi)
        fits = n_cand <= CAP

        def cand_chain(p_lo, p_hi, cb_lo, cb_hi):
            def pc(buf, off):
                @plsc.parallel_loop(0, CHUNK // L, unroll=16, carry=off)
                def loop(v, off):
                    f = buf[pl.ds(v * L, L)]
                    ukey = _monotone_u32_vec(f)
                    m_lo = (ukey ^ p_lo) < jnp.uint32(1 << 21)
                    m_hi = (ukey ^ p_hi) < jnp.uint32(1 << 21)
                    m = jnp.logical_or(m_lo, m_hi)
                    pos = off + plsc.cumsum(m.astype(jnp.int32))
                    plsc.store_scatter(
                        cand, [pos],
                        lax.bitcast_convert_type(ukey, jnp.int32), mask=m)
                    return off + plsc.all_reduce_population_count(m)
                return loop
            stream_row(row, pc, jnp.full((L,), -1, jnp.int32),
                       preprimed=True)

            nvec = (n_cand + (L - 1)) >> 4
            for shift in (14, 7, 0):
                zero_hist(256 * L)

                @plsc.parallel_loop(0, nvec, unroll=4)
                def _(v, shift=shift, p_lo=p_lo, p_hi=p_hi):
                    ukey = lax.bitcast_convert_type(
                        cand[pl.ds(v * L, L)], jnp.uint32)
                    inb = (v * L + lanes) < n_cand
                    hist_update(ukey, shift, 7, p_lo, p_hi, 256, valid=inb)

                p_lo, p_hi, cb_lo, cb_hi, _, _ = scan_both(
                    p_lo, p_hi, cb_lo, cb_hi, shift, 7, 256)
            return p_lo, p_hi, cb_lo, cb_hi

        def full_chain(p_lo, p_hi, cb_lo, cb_hi):
            first = True
            for shift, nbits in ((10, 11), (0, 10)):
                zero_hist(4096 * L)

                def pf(buf, carry, shift=shift, nbits=nbits,
                       p_lo=p_lo, p_hi=p_hi):
                    @plsc.parallel_loop(0, CHUNK // L, unroll=8)
                    def _(v):
                        ukey = _monotone_u32_vec(buf[pl.ds(v * L, L)])
                        hist_update(ukey, shift, nbits, p_lo, p_hi, 4096)
                    return carry
                stream_row(row, pf, jnp.int32(0), preprimed=first)
                first = False
                p_lo, p_hi, cb_lo, cb_hi, _, _ = scan_both(
                    p_lo, p_hi, cb_lo, cb_hi, shift, nbits, 4096)
            return p_lo, p_hi, cb_lo, cb_hi

        p_lo, p_hi, cb_lo, cb_hi = lax.cond(
            fits, cand_chain, full_chain, p_lo, p_hi, cb_lo, cb_hi)

        bot = _u32_to_f32(p_lo)
        top = _u32_to_f32(p_hi)
        mn = jnp.min(minv)
        mx = jnp.max(maxv)
        vec = jnp.where(lanes == 0, bot,
              jnp.where(lanes == 1, top,
              jnp.where(lanes == 2, mn,
              jnp.where(lanes == 3, mx, jnp.float32(0.0)))))
        res[pl.ds(r * L, L)] = vec
        return 0

    lax.fori_loop(0, ROWS_PER_W, row_body, 0)
    pltpu.sync_copy(res, out_hbm.at[pl.ds(wid * ROWS_PER_W * L,
                                          ROWS_PER_W * L)])


def _sc_select(x2d):
    mesh = plsc.VectorSubcoreMesh(core_axis_name="c", subcore_axis_name="s",
                                  num_cores=NC)
    run = functools.partial(
        pl.kernel,
        mesh=mesh,
        compiler_params=pltpu.CompilerParams(needs_layout_passes=False),
        out_type=jax.ShapeDtypeStruct((N_ROWS * L,), jnp.float32),
        scratch_types=[
            pltpu.VMEM((CHUNK,), jnp.float32),
            pltpu.VMEM((CHUNK,), jnp.float32),
            pltpu.VMEM((HWORDS,), jnp.int32),
            pltpu.VMEM((CAP,), jnp.int32),
            pltpu.VMEM((ROWS_PER_W * L,), jnp.float32),
            pltpu.SemaphoreType.DMA,
            pltpu.SemaphoreType.DMA,
        ],
    )(_sc_select_kernel)
    return run(x2d)


def _normalize_body(stats_all_ref, x_ref, stats_row_ref, out_ref):
    s = stats_all_ref[:, 0, :]
    bot_raw, top_raw = s[:, 0], s[:, 1]
    mn, mx = s[:, 2], s[:, 3]
    same = top_raw == bot_raw
    top1 = jnp.where(same, mx, top_raw)
    bot1 = jnp.where(same, mn, bot_raw)
    all_black = jnp.any(top1 == 0.0)
    all_const = jnp.any(top1 == bot1)
    b_row = stats_row_ref[:, 0, 0]
    t_row = stats_row_ref[:, 0, 1]
    same_r = t_row == b_row
    t1 = jnp.where(same_r, stats_row_ref[:, 0, 3], t_row)
    b1 = jnp.where(same_r, stats_row_ref[:, 0, 2], b_row)
    t = jnp.where(all_black, jnp.float32(1.0), t1)
    b = jnp.where(jnp.logical_and(jnp.logical_not(all_black), all_const),
                  jnp.float32(0.0), b1)
    scale = jnp.float32(1.0) / (t - b)
    out_ref[...] = jnp.clip(
        (x_ref[...] - b[:, None, None]) * scale[:, None, None], 0.0, 1.0)


RB = 8


def kernel(x):
    stats = _sc_select(x.reshape(N_ROWS, ROW)).reshape(N_ROWS, 1, L)

    out = pl.pallas_call(
        _normalize_body,
        grid=(N_ROWS // RB,),
        in_specs=[
            pl.BlockSpec((N_ROWS, 1, L), lambda i: (0, 0, 0)),
            pl.BlockSpec((RB, 512, 512), lambda i: (i, 0, 0)),
            pl.BlockSpec((RB, 1, L), lambda i: (i, 0, 0)),
        ],
        out_specs=pl.BlockSpec((RB, 512, 512), lambda i: (i, 0, 0)),
        out_shape=jax.ShapeDtypeStruct((N_ROWS, 512, 512), jnp.float32),
    )(stats, x, stats)
    return out


# speedup vs baseline: 1.8188x; 1.0073x over previous
import functools

import jax
import jax.numpy as jnp
from jax import lax
from jax.experimental import pallas as pl
from jax.experimental.pallas import tpu as pltpu
from jax.experimental.pallas import tpu_sc as plsc

N_ROWS = 96
ROW = 512 * 512
K_BOT = 5243
K_TOP = 256900
NC, NS, L = 2, 16, 16
NW = NC * NS
ROWS_PER_W = N_ROWS // NW
CHUNK = 16384
NCHUNK = ROW // CHUNK
HWORDS = 65536
CAP = 24576
BIG = 1 << 30


def _monotone_u32_vec(f):
    i = lax.bitcast_convert_type(f, jnp.int32)
    flip = lax.shift_right_arithmetic(i, 31) & jnp.int32(0x7FFFFFFF)
    return lax.bitcast_convert_type(i ^ flip, jnp.uint32)


def _u32_to_f32(u):
    i = lax.bitcast_convert_type(u, jnp.int32)
    flip = lax.shift_right_arithmetic(i, 31) & jnp.int32(0x7FFFFFFF)
    return lax.bitcast_convert_type(i ^ flip, jnp.float32)


def _sc_select_kernel(x_hbm, out_hbm, buf0, buf1, hist, cand, res,
                      sem0, sem1):
    wid = lax.axis_index("s") * NC + lax.axis_index("c")
    lanes = lax.broadcasted_iota(jnp.int32, (L,), 0)
    ones_i = jnp.ones((L,), jnp.int32)
    zeros_i = jnp.zeros((L,), jnp.int32)
    true_m = jnp.full((L,), True)

    def start(row, c, buf, sem):
        pltpu.make_async_copy(
            x_hbm.at[row, pl.ds(c * CHUNK, CHUNK)], buf, sem).start()

    def wait(row, buf, sem):
        pltpu.make_async_copy(
            x_hbm.at[row, pl.ds(0, CHUNK)], buf, sem).wait()

    def zero_hist(nwords):
        @plsc.parallel_loop(0, nwords // L, unroll=8)
        def _(i):
            hist[pl.ds(i * L, L)] = zeros_i

    def preprime(row):
        start(row, 0, buf0, sem0)
        start(row, 1, buf1, sem1)

    def stream_row(row, process, carry, preprimed=False):
        if not preprimed:
            preprime(row)

        def cbody(g, carry):
            c0 = 2 * g
            wait(row, buf0, sem0)
            carry = process(buf0, carry)

            @pl.when(c0 + 2 < NCHUNK)
            def _():
                start(row, c0 + 2, buf0, sem0)
            wait(row, buf1, sem1)
            carry = process(buf1, carry)

            @pl.when(c0 + 3 < NCHUNK)
            def _():
                start(row, c0 + 3, buf1, sem1)
            return carry

        return lax.fori_loop(0, NCHUNK // 2, cbody, carry)

    def hist_update(ukey, shift, nbits, p_lo, p_hi, stride, valid=None):
        digit = (lax.shift_right_logical(ukey, jnp.uint32(shift))
                 .astype(jnp.int32)) & jnp.int32((1 << nbits) - 1)
        m_lo = (ukey ^ p_lo) < jnp.uint32(1 << (shift + nbits))
        m_hi = (ukey ^ p_hi) < jnp.uint32(1 << (shift + nbits))
        if valid is not None:
            m_lo = jnp.logical_and(m_lo, valid)
            m_hi = jnp.logical_and(m_hi, valid)
        idx = lanes * stride + digit + jnp.where(
            m_hi, jnp.int32(stride // 2), jnp.int32(0))
        plsc.addupdate_scatter(hist, [idx], ones_i,
                               mask=jnp.logical_or(m_lo, m_hi))

    def scan_region(base, kplus1, ndig, stride):
        def sbody(g, carry):
            csum, cnt, below, upper = carry
            acc = zeros_i
            for l in range(L):
                acc = acc + hist[pl.ds(l * stride + base + g * L, L)]
            pre = plsc.cumsum(acc) + csum
            lt = pre < kplus1
            cnt = cnt + jnp.sum(lt.astype(jnp.int32))
            below = jnp.maximum(below, jnp.max(jnp.where(lt, pre, 0)))
            upper = jnp.minimum(upper, jnp.min(
                jnp.where(lt, jnp.int32(BIG), pre)))
            csum = csum + jnp.sum(acc)
            return csum, cnt, below, upper
        _, digit, below, upper = lax.fori_loop(
            0, ndig // L, sbody,
            (jnp.int32(0), jnp.int32(0), jnp.int32(0), jnp.int32(BIG)))
        return digit, below, upper - below

    def scan_both(p_lo, p_hi, cb_lo, cb_hi, shift, nbits, stride):
        ndig = 1 << nbits
        if shift + nbits == 32:
            base_lo = jnp.int32(0)
            base_hi = jnp.int32(0)
        else:
            eq = (p_lo >> jnp.uint32(shift + nbits)) == (
                p_hi >> jnp.uint32(shift + nbits))
            base_lo = jnp.where(eq, jnp.int32(stride // 2), jnp.int32(0))
            base_hi = jnp.int32(stride // 2)
        d_lo, below_lo, bk_lo = scan_region(
            base_lo, K_BOT + 1 - cb_lo, ndig, stride)
        d_hi, below_hi, bk_hi = scan_region(
            base_hi, K_TOP + 1 - cb_hi, ndig, stride)
        p_lo = p_lo | lax.shift_left(d_lo.astype(jnp.uint32),
                                     jnp.uint32(shift))
        p_hi = p_hi | lax.shift_left(d_hi.astype(jnp.uint32),
                                     jnp.uint32(shift))
        return p_lo, p_hi, cb_lo + below_lo, cb_hi + below_hi, bk_lo, bk_hi

    def row_body(r, _):
        row = wid * ROWS_PER_W + r

        zero_hist(2048 * L)

        def p0(buf, carry):
            @plsc.parallel_loop(0, CHUNK // L, unroll=16, carry=carry)
            def mm(v, carry):
                mnv, mxv = carry
                f = buf[pl.ds(v * L, L)]
                ukey = _monotone_u32_vec(f)
                digit = lax.shift_right_logical(
                    ukey, jnp.uint32(21)).astype(jnp.int32)
                plsc.addupdate_scatter(hist, [lanes * 2048 + digit],
                                       ones_i, mask=true_m)
                return jnp.minimum(mnv, f), jnp.maximum(mxv, f)
            return mm

        minv, maxv = stream_row(row, p0, (
            jnp.full((L,), jnp.float32(jnp.inf)),
            jnp.full((L,), -jnp.float32(jnp.inf))), preprimed=True)
        preprime(row)

        def s0body(g, carry):
            (csum, cnt1, below1, upper1, cnt2, below2, upper2) = carry
            gw = ((g + 64) & 127) * L
            acc = zeros_i
            for l in range(L):
                acc = acc + hist[pl.ds(l * 2048 + gw, L)]
            pre = plsc.cumsum(acc) + csum
            lt1 = pre < K_BOT + 1
            cnt1 = cnt1 + lt1.astype(jnp.int32)
            below1 = jnp.maximum(below1, jnp.where(lt1, pre, 0))
            upper1 = jnp.minimum(upper1, jnp.where(lt1, jnp.int32(BIG), pre))
            lt2 = pre < K_TOP + 1
            cnt2 = cnt2 + lt2.astype(jnp.int32)
            below2 = jnp.maximum(below2, jnp.where(lt2, pre, 0))
            upper2 = jnp.minimum(upper2, jnp.where(lt2, jnp.int32(BIG), pre))
            csum = csum + jnp.sum(acc)
            return (csum, cnt1, below1, upper1, cnt2, below2, upper2)

        zv = zeros_i
        bigv = jnp.full((L,), jnp.int32(BIG))
        (_, c1v, b1v, u1v, c2v, b2v, u2v) = lax.fori_loop(
            0, 128, s0body, (jnp.int32(0), zv, zv, bigv, zv, zv, bigv))
        c1 = jnp.sum(c1v)
        cb_lo = jnp.max(b1v)
        u1 = jnp.min(u1v)
        c2 = jnp.sum(c2v)
        cb_hi = jnp.max(b2v)
        u2 = jnp.min(u2v)
        bk_lo = u1 - cb_lo
        bk_hi = u2 - cb_hi
        d_lo = (1024 + c1) & 2047
        d_hi = (1024 + c2) & 2047
        p_lo = lax.shift_left(d_lo.astype(jnp.uint32), jnp.uint32(21))
        p_hi = lax.shift_left(d_hi.astype(jnp.uint32), jnp.uint32(21))

        eq11 = p_lo == p_hi
        n_cand = jnp.where(eq11, bk_lo, bk_lo + bk_hi)
        fits = n_cand <= CAP

        def cand_chain(p_lo, p_hi, cb_lo, cb_hi):
            def pc(buf, off):
                @plsc.parallel_loop(0, CHUNK // L, unroll=16, carry=off)
                def loop(v, off):
                    f = buf[pl.ds(v * L, L)]
                    ukey = _monotone_u32_vec(f)
                    m_lo = (ukey ^ p_lo) < jnp.uint32(1 << 21)
                    m_hi = (ukey ^ p_hi) < jnp.uint32(1 << 21)
                    m = jnp.logical_or(m_lo, m_hi)
                    pos = off + plsc.cumsum(m.astype(jnp.int32))
                    plsc.store_scatter(
                        cand, [pos],
                        lax.bitcast_convert_type(ukey, jnp.int32), mask=m)
                    return off + plsc.all_reduce_population_count(m)
                return loop
            stream_row(row, pc, jnp.full((L,), -1, jnp.int32),
                       preprimed=True)

            nvec = (n_cand + (L - 1)) >> 4
            for shift in (14, 7, 0):
                zero_hist(256 * L)

                @plsc.parallel_loop(0, nvec, unroll=4)
                def _(v, shift=shift, p_lo=p_lo, p_hi=p_hi):
                    ukey = lax.bitcast_convert_type(
                        cand[pl.ds(v * L, L)], jnp.uint32)
                    inb = (v * L + lanes) < n_cand
                    hist_update(ukey, shift, 7, p_lo, p_hi, 256, valid=inb)

                p_lo, p_hi, cb_lo, cb_hi, _, _ = scan_both(
                    p_lo, p_hi, cb_lo, cb_hi, shift, 7, 256)
            return p_lo, p_hi, cb_lo, cb_hi

        def full_chain(p_lo, p_hi, cb_lo, cb_hi):
            first = True
            for shift, nbits in ((10, 11), (0, 10)):
                zero_hist(4096 * L)

                def pf(buf, carry, shift=shift, nbits=nbits,
                       p_lo=p_lo, p_hi=p_hi):
                    @plsc.parallel_loop(0, CHUNK // L, unroll=8)
                    def _(v):
                        ukey = _monotone_u32_vec(buf[pl.ds(v * L, L)])
                        hist_update(ukey, shift, nbits, p_lo, p_hi, 4096)
                    return carry
                stream_row(row, pf, jnp.int32(0), preprimed=first)
                first = False
                p_lo, p_hi, cb_lo, cb_hi, _, _ = scan_both(
                    p_lo, p_hi, cb_lo, cb_hi, shift, nbits, 4096)
            return p_lo, p_hi, cb_lo, cb_hi

        p_lo, p_hi, cb_lo, cb_hi = lax.cond(
            fits, cand_chain, full_chain, p_lo, p_hi, cb_lo, cb_hi)

        bot = _u32_to_f32(p_lo)
        top = _u32_to_f32(p_hi)
        mn = jnp.min(minv)
        mx = jnp.max(maxv)
        vec = jnp.where(lanes == 0, bot,
              jnp.where(lanes == 1, top,
              jnp.where(lanes == 2, mn,
              jnp.where(lanes == 3, mx, jnp.float32(0.0)))))
        res[pl.ds(r * L, L)] = vec

        @pl.when(r < ROWS_PER_W - 1)
        def _():
            preprime(row + 1)
        return 0

    preprime(wid * ROWS_PER_W)
    lax.fori_loop(0, ROWS_PER_W, row_body, 0)
    pltpu.sync_copy(res, out_hbm.at[pl.ds(wid * ROWS_PER_W * L,
                                          ROWS_PER_W * L)])


def _sc_select(x2d):
    mesh = plsc.VectorSubcoreMesh(core_axis_name="c", subcore_axis_name="s",
                                  num_cores=NC)
    run = functools.partial(
        pl.kernel,
        mesh=mesh,
        compiler_params=pltpu.CompilerParams(needs_layout_passes=False),
        out_type=jax.ShapeDtypeStruct((N_ROWS * L,), jnp.float32),
        scratch_types=[
            pltpu.VMEM((CHUNK,), jnp.float32),
            pltpu.VMEM((CHUNK,), jnp.float32),
            pltpu.VMEM((HWORDS,), jnp.int32),
            pltpu.VMEM((CAP,), jnp.int32),
            pltpu.VMEM((ROWS_PER_W * L,), jnp.float32),
            pltpu.SemaphoreType.DMA,
            pltpu.SemaphoreType.DMA,
        ],
    )(_sc_select_kernel)
    return run(x2d)


def _normalize_body(stats_all_ref, x_ref, stats_row_ref, out_ref):
    s = stats_all_ref[:, 0, :]
    bot_raw, top_raw = s[:, 0], s[:, 1]
    mn, mx = s[:, 2], s[:, 3]
    same = top_raw == bot_raw
    top1 = jnp.where(same, mx, top_raw)
    bot1 = jnp.where(same, mn, bot_raw)
    all_black = jnp.any(top1 == 0.0)
    all_const = jnp.any(top1 == bot1)
    b_row = stats_row_ref[:, 0, 0]
    t_row = stats_row_ref[:, 0, 1]
    same_r = t_row == b_row
    t1 = jnp.where(same_r, stats_row_ref[:, 0, 3], t_row)
    b1 = jnp.where(same_r, stats_row_ref[:, 0, 2], b_row)
    t = jnp.where(all_black, jnp.float32(1.0), t1)
    b = jnp.where(jnp.logical_and(jnp.logical_not(all_black), all_const),
                  jnp.float32(0.0), b1)
    scale = jnp.float32(1.0) / (t - b)
    out_ref[...] = jnp.clip(
        (x_ref[...] - b[:, None, None]) * scale[:, None, None], 0.0, 1.0)


RB = 8


def kernel(x):
    stats = _sc_select(x.reshape(N_ROWS, ROW)).reshape(N_ROWS, 1, L)

    out = pl.pallas_call(
        _normalize_body,
        grid=(N_ROWS // RB,),
        in_specs=[
            pl.BlockSpec((N_ROWS, 1, L), lambda i: (0, 0, 0)),
            pl.BlockSpec((RB, 512, 512), lambda i: (i, 0, 0)),
            pl.BlockSpec((RB, 1, L), lambda i: (i, 0, 0)),
        ],
        out_specs=pl.BlockSpec((RB, 512, 512), lambda i: (i, 0, 0)),
        out_shape=jax.ShapeDtypeStruct((N_ROWS, 512, 512), jnp.float32),
    )(stats, x, stats)
    return out
